# Initial kernel scaffold; baseline (speedup 1.0000x reference)
import jax
import jax.numpy as jnp
from jax.experimental import pallas as pl


def kernel(x, W1, b1, W2, b2, W5, b5):
    raise NotImplementedError("write your pallas kernel here")


# speedup vs baseline: 14.7130x; 14.7130x over previous
import functools

import jax
import jax.numpy as jnp
from jax import lax
from jax.experimental import pallas as pl
from jax.experimental.pallas import tpu as pltpu
from jax.experimental.pallas import tpu_sc as plsc

B, N, K = 4, 4096, 16
M = B * N
ROWS = 256
NEG = float("-inf")


def _knn_body(xt_ref, xall_ref, wnt_ref, wct_ref, bias_ref,
              idx_ref, u_ref, v_ref):
    b = pl.program_id(0)
    rows = xt_ref[0]
    alln = xall_ref[0]
    inner2 = 2.0 * lax.dot_general(
        rows, alln, (((1,), (0,)), ((), ())),
        preferred_element_type=jnp.float32)
    rowsq = jnp.sum(rows * rows, axis=1, keepdims=True)
    colsq = jnp.sum(alln * alln, axis=0, keepdims=True)
    d = inner2 - rowsq - colsq

    neg_iota = lax.broadcasted_iota(jnp.int32, d.shape, 1).astype(jnp.float32) * (-1.0)
    cols = []
    for t in range(K):
        m = jnp.max(d, axis=1, keepdims=True)
        hit = d == m
        am = jnp.max(jnp.where(hit, neg_iota, NEG), axis=1, keepdims=True)
        cols.append((-am).astype(jnp.int32))
        if t + 1 < K:
            d = jnp.where(hit, NEG, d)
    idx_ref[0] = jnp.concatenate(cols, axis=1) + b * N

    u = lax.dot_general(rows, wnt_ref[...], (((1,), (0,)), ((), ())),
                        preferred_element_type=jnp.float32)
    u_ref[0] = jnp.concatenate([u, u], axis=1)
    v_ref[0] = lax.dot_general(rows, wct_ref[...], (((1,), (0,)), ((), ())),
                               preferred_element_type=jnp.float32) + bias_ref[...]


def _knn(xt, xall, wnt, wct, bias):
    c = xt.shape[-1]
    grid = (B, N // ROWS)
    return pl.pallas_call(
        _knn_body,
        grid=grid,
        in_specs=[
            pl.BlockSpec((1, ROWS, c), lambda b, i: (b, i, 0)),
            pl.BlockSpec((1, c, N), lambda b, i: (b, 0, 0)),
            pl.BlockSpec((c, 64), lambda b, i: (0, 0)),
            pl.BlockSpec((c, 64), lambda b, i: (0, 0)),
            pl.BlockSpec((1, 64), lambda b, i: (0, 0)),
        ],
        out_specs=[
            pl.BlockSpec((1, ROWS, K), lambda b, i: (b, i, 0)),
            pl.BlockSpec((1, ROWS, 128), lambda b, i: (b, i, 0)),
            pl.BlockSpec((1, ROWS, 64), lambda b, i: (b, i, 0)),
        ],
        out_shape=[
            jax.ShapeDtypeStruct((B, N, K), jnp.int32),
            jax.ShapeDtypeStruct((B, N, 128), jnp.float32),
            jax.ShapeDtypeStruct((B, N, 64), jnp.float32),
        ],
    )(xt, xall, wnt, wct, bias)


_NC, _NS = 2, 16
_NW = _NC * _NS
_PW = M // _NW
_P = 8
_NG = _PW // _P


def _gmax_body(u_hbm, idx_hbm, v_hbm, o_hbm, idx_v, rows_v, v_v, o_v, sem):
    wid = lax.axis_index("s") * _NC + lax.axis_index("c")
    base = wid * _PW

    def step(g, carry):
        pbase = base + g * _P
        pltpu.sync_copy(idx_hbm.at[pl.ds(pbase * K, _P * K)], idx_v)
        pltpu.async_copy(u_hbm.at[idx_v], rows_v, sem).wait()
        pltpu.sync_copy(v_hbm.at[pl.ds(pbase, _P)], v_v)
        for p in range(_P):
            for cch in range(4):
                sl = pl.ds(cch * 16, 16)
                acc = rows_v[p * K, sl]
                for r in range(1, K):
                    acc = jnp.maximum(acc, rows_v[p * K + r, sl])
                z = acc + v_v[p, sl]
                o_v[p, sl] = jnp.maximum(z, 0.2 * z)
        pltpu.sync_copy(o_v, o_hbm.at[pl.ds(pbase, _P)])
        return carry

    lax.fori_loop(0, _NG, step, 0)


def _gmax(u_flat, idx_flat, v_flat):
    mesh = plsc.VectorSubcoreMesh(core_axis_name="c", subcore_axis_name="s")
    f = functools.partial(
        pl.kernel,
        mesh=mesh,
        out_type=jax.ShapeDtypeStruct((M, 64), jnp.float32),
        scratch_types=[
            pltpu.VMEM((_P * K,), jnp.int32),
            pltpu.VMEM((_P * K, 128), jnp.float32),
            pltpu.VMEM((_P, 64), jnp.float32),
            pltpu.VMEM((_P, 64), jnp.float32),
            pltpu.SemaphoreType.DMA,
        ],
    )(_gmax_body)
    return f(u_flat, idx_flat, v_flat)


_CB = 1024


def _final_body(x1_ref, x2_ref, w5a_ref, w5b_ref, b5_ref, o_ref):
    h = (lax.dot_general(w5a_ref[...], x1_ref[0], (((1,), (0,)), ((), ())),
                         preferred_element_type=jnp.float32)
         + lax.dot_general(w5b_ref[...], x2_ref[0], (((1,), (0,)), ((), ())),
                           preferred_element_type=jnp.float32)
         + b5_ref[...])
    o_ref[0] = jnp.maximum(h, 0.2 * h)


def _final(x1t, x2t, w5a, w5b, b5col):
    grid = (B, N // _CB)
    return pl.pallas_call(
        _final_body,
        grid=grid,
        in_specs=[
            pl.BlockSpec((1, 64, _CB), lambda b, i: (b, 0, i)),
            pl.BlockSpec((1, 64, _CB), lambda b, i: (b, 0, i)),
            pl.BlockSpec((512, 64), lambda b, i: (0, 0)),
            pl.BlockSpec((512, 64), lambda b, i: (0, 0)),
            pl.BlockSpec((512, 1), lambda b, i: (0, 0)),
        ],
        out_specs=pl.BlockSpec((1, 512, _CB), lambda b, i: (b, 0, i)),
        out_shape=jax.ShapeDtypeStruct((B, 512, N), jnp.float32),
    )(x1t, x2t, w5a, w5b, b5col)


def kernel(x, W1, b1, W2, b2, W5, b5):
    xt = jnp.transpose(x, (0, 2, 1))
    xt8 = jnp.pad(xt, ((0, 0), (0, 0), (0, 5)))
    x8 = jnp.pad(x, ((0, 0), (0, 5), (0, 0)))
    w1t = jnp.transpose(W1)
    w1nt = jnp.pad(w1t[:3], ((0, 5), (0, 0)))
    w1ct = jnp.pad(w1t[3:], ((0, 5), (0, 0)))
    idx1, u1, v1 = _knn(xt8, x8, w1nt, w1ct, b1.reshape(1, 64))
    x1 = _gmax(u1.reshape(M, 128), idx1.reshape(M * K), v1.reshape(M, 64))
    x1 = x1.reshape(B, N, 64)
    x1t = jnp.transpose(x1, (0, 2, 1))

    w2t = jnp.transpose(W2)
    idx2, u2, v2 = _knn(x1, x1t, w2t[:64], w2t[64:], b2.reshape(1, 64))
    x2 = _gmax(u2.reshape(M, 128), idx2.reshape(M * K), v2.reshape(M, 64))
    x2t = jnp.transpose(x2.reshape(B, N, 64), (0, 2, 1))

    return _final(x1t, x2t, W5[:, :64], W5[:, 64:], b5.reshape(512, 1))


# speedup vs baseline: 18.1436x; 1.2332x over previous
import functools

import jax
import jax.numpy as jnp
from jax import lax
from jax.experimental import pallas as pl
from jax.experimental.pallas import tpu as pltpu
from jax.experimental.pallas import tpu_sc as plsc

B, N, K = 4, 4096, 16
M = B * N
ROWS = 256
NEG = float("-inf")


def _knn_body(xt_ref, xall_ref, wnt_ref, wct_ref, bias_ref,
              idx_ref, u_ref, v_ref):
    b = pl.program_id(0)
    rows = xt_ref[0]
    alln = xall_ref[0]
    inner2 = 2.0 * lax.dot_general(
        rows, alln, (((1,), (0,)), ((), ())),
        preferred_element_type=jnp.float32)
    rowsq = jnp.sum(rows * rows, axis=1, keepdims=True)
    colsq = jnp.sum(alln * alln, axis=0, keepdims=True)
    d = inner2 - rowsq - colsq

    bi = lax.bitcast_convert_type(d, jnp.int32)
    skey = bi ^ (lax.shift_right_arithmetic(bi, 31) & jnp.int32(0x7FFFFFFF))
    vkey = lax.shift_right_arithmetic(skey, 1) + jnp.int32(0x40000000)
    iota_i = lax.broadcasted_iota(jnp.int32, d.shape, 1)
    fkeys = lax.bitcast_convert_type(
        (vkey & jnp.int32(-4096)) | (jnp.int32(N - 1) - iota_i), jnp.float32)
    cols = []
    for t in range(K):
        m = jnp.max(fkeys, axis=1, keepdims=True)
        mb = lax.bitcast_convert_type(m, jnp.int32)
        cols.append(jnp.int32(N - 1) - (mb & jnp.int32(4095)))
        fkeys = jnp.where(fkeys == m, NEG, fkeys)
    m17 = jnp.max(fkeys, axis=1, keepdims=True)
    mb17 = lax.bitcast_convert_type(m17, jnp.int32)
    col17 = jnp.int32(N - 1) - (mb17 & jnp.int32(4095))
    e16 = jnp.max(jnp.where(iota_i == cols[K - 1], d, NEG), axis=1, keepdims=True)
    e17 = jnp.max(jnp.where(iota_i == col17, d, NEG), axis=1, keepdims=True)
    cols[K - 1] = jnp.where(e17 > e16, col17, cols[K - 1])
    idx_ref[0] = jnp.concatenate(cols, axis=1) + b * N

    u = lax.dot_general(rows, wnt_ref[...], (((1,), (0,)), ((), ())),
                        preferred_element_type=jnp.float32)
    u_ref[0] = jnp.concatenate([u, u], axis=1)
    v_ref[0] = lax.dot_general(rows, wct_ref[...], (((1,), (0,)), ((), ())),
                               preferred_element_type=jnp.float32) + bias_ref[...]


def _knn(xt, xall, wnt, wct, bias):
    c = xt.shape[-1]
    grid = (B, N // ROWS)
    return pl.pallas_call(
        _knn_body,
        grid=grid,
        in_specs=[
            pl.BlockSpec((1, ROWS, c), lambda b, i: (b, i, 0)),
            pl.BlockSpec((1, c, N), lambda b, i: (b, 0, 0)),
            pl.BlockSpec((c, 64), lambda b, i: (0, 0)),
            pl.BlockSpec((c, 64), lambda b, i: (0, 0)),
            pl.BlockSpec((1, 64), lambda b, i: (0, 0)),
        ],
        out_specs=[
            pl.BlockSpec((1, ROWS, K), lambda b, i: (b, i, 0)),
            pl.BlockSpec((1, ROWS, 128), lambda b, i: (b, i, 0)),
            pl.BlockSpec((1, ROWS, 64), lambda b, i: (b, i, 0)),
        ],
        out_shape=[
            jax.ShapeDtypeStruct((B, N, K), jnp.int32),
            jax.ShapeDtypeStruct((B, N, 128), jnp.float32),
            jax.ShapeDtypeStruct((B, N, 64), jnp.float32),
        ],
    )(xt, xall, wnt, wct, bias)


_NC, _NS = 2, 16
_NW = _NC * _NS
_PW = M // _NW
_P = 8
_NG = _PW // _P


def _gmax_body(u_hbm, idx_hbm, v_hbm, o_hbm, idx_v, rows_v, v_v, o_v, sem):
    wid = lax.axis_index("s") * _NC + lax.axis_index("c")
    base = wid * _PW

    def step(g, carry):
        pbase = base + g * _P
        pltpu.sync_copy(idx_hbm.at[pl.ds(pbase * K, _P * K)], idx_v)
        pltpu.async_copy(u_hbm.at[idx_v], rows_v, sem).wait()
        pltpu.sync_copy(v_hbm.at[pl.ds(pbase, _P)], v_v)
        for p in range(_P):
            for cch in range(4):
                sl = pl.ds(cch * 16, 16)
                acc = rows_v[p * K, sl]
                for r in range(1, K):
                    acc = jnp.maximum(acc, rows_v[p * K + r, sl])
                z = acc + v_v[p, sl]
                o_v[p, sl] = jnp.maximum(z, 0.2 * z)
        pltpu.sync_copy(o_v, o_hbm.at[pl.ds(pbase, _P)])
        return carry

    lax.fori_loop(0, _NG, step, 0)


def _gmax(u_flat, idx_flat, v_flat):
    mesh = plsc.VectorSubcoreMesh(core_axis_name="c", subcore_axis_name="s")
    f = functools.partial(
        pl.kernel,
        mesh=mesh,
        out_type=jax.ShapeDtypeStruct((M, 64), jnp.float32),
        scratch_types=[
            pltpu.VMEM((_P * K,), jnp.int32),
            pltpu.VMEM((_P * K, 128), jnp.float32),
            pltpu.VMEM((_P, 64), jnp.float32),
            pltpu.VMEM((_P, 64), jnp.float32),
            pltpu.SemaphoreType.DMA,
        ],
    )(_gmax_body)
    return f(u_flat, idx_flat, v_flat)


_CB = 1024


def _final_body(x1_ref, x2_ref, w5a_ref, w5b_ref, b5_ref, o_ref):
    h = (lax.dot_general(w5a_ref[...], x1_ref[0], (((1,), (0,)), ((), ())),
                         preferred_element_type=jnp.float32)
         + lax.dot_general(w5b_ref[...], x2_ref[0], (((1,), (0,)), ((), ())),
                           preferred_element_type=jnp.float32)
         + b5_ref[...])
    o_ref[0] = jnp.maximum(h, 0.2 * h)


def _final(x1t, x2t, w5a, w5b, b5col):
    grid = (B, N // _CB)
    return pl.pallas_call(
        _final_body,
        grid=grid,
        in_specs=[
            pl.BlockSpec((1, 64, _CB), lambda b, i: (b, 0, i)),
            pl.BlockSpec((1, 64, _CB), lambda b, i: (b, 0, i)),
            pl.BlockSpec((512, 64), lambda b, i: (0, 0)),
            pl.BlockSpec((512, 64), lambda b, i: (0, 0)),
            pl.BlockSpec((512, 1), lambda b, i: (0, 0)),
        ],
        out_specs=pl.BlockSpec((1, 512, _CB), lambda b, i: (b, 0, i)),
        out_shape=jax.ShapeDtypeStruct((B, 512, N), jnp.float32),
    )(x1t, x2t, w5a, w5b, b5col)


def kernel(x, W1, b1, W2, b2, W5, b5):
    xt = jnp.transpose(x, (0, 2, 1))
    xt8 = jnp.pad(xt, ((0, 0), (0, 0), (0, 5)))
    x8 = jnp.pad(x, ((0, 0), (0, 5), (0, 0)))
    w1t = jnp.transpose(W1)
    w1nt = jnp.pad(w1t[:3], ((0, 5), (0, 0)))
    w1ct = jnp.pad(w1t[3:], ((0, 5), (0, 0)))
    idx1, u1, v1 = _knn(xt8, x8, w1nt, w1ct, b1.reshape(1, 64))
    x1 = _gmax(u1.reshape(M, 128), idx1.reshape(M * K), v1.reshape(M, 64))
    x1 = x1.reshape(B, N, 64)
    x1t = jnp.transpose(x1, (0, 2, 1))

    w2t = jnp.transpose(W2)
    idx2, u2, v2 = _knn(x1, x1t, w2t[:64], w2t[64:], b2.reshape(1, 64))
    x2 = _gmax(u2.reshape(M, 128), idx2.reshape(M * K), v2.reshape(M, 64))
    x2t = jnp.transpose(x2.reshape(B, N, 64), (0, 2, 1))

    return _final(x1t, x2t, W5[:, :64], W5[:, 64:], b5.reshape(512, 1))


# speedup vs baseline: 20.8158x; 1.1473x over previous
import functools

import jax
import jax.numpy as jnp
from jax import lax
from jax.experimental import pallas as pl
from jax.experimental.pallas import tpu as pltpu
from jax.experimental.pallas import tpu_sc as plsc

B, N, K = 4, 4096, 16
M = B * N
ROWS = 256
NEG = float("-inf")


def _knn_body(xt_ref, xall_ref, wnt_ref, wct_ref, bias_ref,
              idx_ref, u_ref, v_ref):
    b = pl.program_id(0)
    rows = xt_ref[0]
    alln = xall_ref[0]
    inner2 = 2.0 * lax.dot_general(
        rows, alln, (((1,), (0,)), ((), ())),
        preferred_element_type=jnp.float32)
    rowsq = jnp.sum(rows * rows, axis=1, keepdims=True)
    colsq = jnp.sum(alln * alln, axis=0, keepdims=True)
    d = inner2 - rowsq - colsq

    bi = lax.bitcast_convert_type(d, jnp.int32)
    skey = bi ^ (lax.shift_right_arithmetic(bi, 31) & jnp.int32(0x7FFFFFFF))
    vkey = lax.shift_right_arithmetic(skey, 1) + jnp.int32(0x40000000)
    iota_i = lax.broadcasted_iota(jnp.int32, d.shape, 1)
    fkeys = lax.bitcast_convert_type(
        (vkey & jnp.int32(-4096)) | (jnp.int32(N - 1) - iota_i), jnp.float32)
    cols = []
    for t in range(K):
        m = jnp.max(fkeys, axis=1, keepdims=True)
        mb = lax.bitcast_convert_type(m, jnp.int32)
        cols.append(jnp.int32(N - 1) - (mb & jnp.int32(4095)))
        fkeys = jnp.where(fkeys == m, NEG, fkeys)
    m17 = jnp.max(fkeys, axis=1, keepdims=True)
    mb17 = lax.bitcast_convert_type(m17, jnp.int32)
    col17 = jnp.int32(N - 1) - (mb17 & jnp.int32(4095))
    e16 = jnp.max(jnp.where(iota_i == cols[K - 1], d, NEG), axis=1, keepdims=True)
    e17 = jnp.max(jnp.where(iota_i == col17, d, NEG), axis=1, keepdims=True)
    cols[K - 1] = jnp.where(e17 > e16, col17, cols[K - 1])
    idx_ref[0] = jnp.concatenate(cols, axis=1) + b * N

    u = lax.dot_general(rows, wnt_ref[...], (((1,), (0,)), ((), ())),
                        preferred_element_type=jnp.float32)
    u_ref[0] = jnp.concatenate([u, u], axis=1)
    v_ref[0] = lax.dot_general(rows, wct_ref[...], (((1,), (0,)), ((), ())),
                               preferred_element_type=jnp.float32) + bias_ref[...]


def _knn(xt, xall, wnt, wct, bias):
    c = xt.shape[-1]
    grid = (B, N // ROWS)
    return pl.pallas_call(
        _knn_body,
        grid=grid,
        in_specs=[
            pl.BlockSpec((1, ROWS, c), lambda b, i: (b, i, 0)),
            pl.BlockSpec((1, c, N), lambda b, i: (b, 0, 0)),
            pl.BlockSpec((c, 64), lambda b, i: (0, 0)),
            pl.BlockSpec((c, 64), lambda b, i: (0, 0)),
            pl.BlockSpec((1, 64), lambda b, i: (0, 0)),
        ],
        out_specs=[
            pl.BlockSpec((1, ROWS, K), lambda b, i: (b, i, 0)),
            pl.BlockSpec((1, ROWS, 128), lambda b, i: (b, i, 0)),
            pl.BlockSpec((1, ROWS, 64), lambda b, i: (b, i, 0)),
        ],
        out_shape=[
            jax.ShapeDtypeStruct((B, N, K), jnp.int32),
            jax.ShapeDtypeStruct((B, N, 128), jnp.float32),
            jax.ShapeDtypeStruct((B, N, 64), jnp.float32),
        ],
    )(xt, xall, wnt, wct, bias)


_NC, _NS = 2, 16
_NW = _NC * _NS
_PW = M // _NW
_P = 8
_NG = _PW // _P


def _gmax_body(u_hbm, idx_hbm, v_hbm, o_hbm, idx_v, rows_v, v_v, o_v,
               sg0, sg1, so0, so1):
    wid = lax.axis_index("s") * _NC + lax.axis_index("c")
    base = wid * _PW
    sg = (sg0, sg1)
    so = (so0, so1)

    pltpu.sync_copy(idx_hbm.at[pl.ds(wid * _NG, _NG)], idx_v)

    def fire(g, slot):
        pltpu.async_copy(u_hbm.at[idx_v.at[g]], rows_v.at[slot], sg[slot])
        pltpu.async_copy(v_hbm.at[pl.ds(base + g * _P, _P)], v_v.at[slot],
                         sg[slot])

    def drain(g, slot):
        pltpu.make_async_copy(u_hbm.at[idx_v.at[g]], rows_v.at[slot],
                              sg[slot]).wait()
        pltpu.make_async_copy(v_hbm.at[pl.ds(base + g * _P, _P)],
                              v_v.at[slot], sg[slot]).wait()

    def drain_out(g, slot):
        pltpu.make_async_copy(o_v.at[slot],
                              o_hbm.at[pl.ds(base + g * _P, _P)],
                              so[slot]).wait()

    fire(0, 0)

    def step(g2, carry):
        for slot in (0, 1):
            g = g2 * 2 + slot
            nslot = 1 - slot

            @pl.when(g + 1 < _NG)
            def _():
                fire(g + 1, nslot)

            drain(g, slot)

            @pl.when(g >= 2)
            def _():
                drain_out(g - 2, slot)

            for p in range(_P):
                for cch in range(4):
                    sl = pl.ds(cch * 16, 16)
                    acc = rows_v[slot, p * K, sl]
                    for r in range(1, K):
                        acc = jnp.maximum(acc, rows_v[slot, p * K + r, sl])
                    z = acc + v_v[slot, p, sl]
                    o_v[slot, p, sl] = jnp.maximum(z, 0.2 * z)
            pltpu.async_copy(o_v.at[slot], o_hbm.at[pl.ds(base + g * _P, _P)],
                             so[slot])
        return carry

    lax.fori_loop(0, _NG // 2, step, 0)
    drain_out(_NG - 2, 0)
    drain_out(_NG - 1, 1)


def _gmax(u_flat, idx_rows, v_flat):
    mesh = plsc.VectorSubcoreMesh(core_axis_name="c", subcore_axis_name="s")
    f = functools.partial(
        pl.kernel,
        mesh=mesh,
        out_type=jax.ShapeDtypeStruct((M, 64), jnp.float32),
        scratch_types=[
            pltpu.VMEM((_NG, _P * K), jnp.int32),
            pltpu.VMEM((2, _P * K, 128), jnp.float32),
            pltpu.VMEM((2, _P, 64), jnp.float32),
            pltpu.VMEM((2, _P, 64), jnp.float32),
            pltpu.SemaphoreType.DMA,
            pltpu.SemaphoreType.DMA,
            pltpu.SemaphoreType.DMA,
            pltpu.SemaphoreType.DMA,
        ],
    )(_gmax_body)
    return f(u_flat, idx_rows, v_flat)


_CB = 1024


def _final_body(x1_ref, x2_ref, w5a_ref, w5b_ref, b5_ref, o_ref):
    h = (lax.dot_general(w5a_ref[...], x1_ref[0], (((1,), (0,)), ((), ())),
                         preferred_element_type=jnp.float32)
         + lax.dot_general(w5b_ref[...], x2_ref[0], (((1,), (0,)), ((), ())),
                           preferred_element_type=jnp.float32)
         + b5_ref[...])
    o_ref[0] = jnp.maximum(h, 0.2 * h)


def _final(x1t, x2t, w5a, w5b, b5col):
    grid = (B, N // _CB)
    return pl.pallas_call(
        _final_body,
        grid=grid,
        in_specs=[
            pl.BlockSpec((1, 64, _CB), lambda b, i: (b, 0, i)),
            pl.BlockSpec((1, 64, _CB), lambda b, i: (b, 0, i)),
            pl.BlockSpec((512, 64), lambda b, i: (0, 0)),
            pl.BlockSpec((512, 64), lambda b, i: (0, 0)),
            pl.BlockSpec((512, 1), lambda b, i: (0, 0)),
        ],
        out_specs=pl.BlockSpec((1, 512, _CB), lambda b, i: (b, 0, i)),
        out_shape=jax.ShapeDtypeStruct((B, 512, N), jnp.float32),
    )(x1t, x2t, w5a, w5b, b5col)


def kernel(x, W1, b1, W2, b2, W5, b5):
    xt = jnp.transpose(x, (0, 2, 1))
    xt8 = jnp.pad(xt, ((0, 0), (0, 0), (0, 5)))
    x8 = jnp.pad(x, ((0, 0), (0, 5), (0, 0)))
    w1t = jnp.transpose(W1)
    w1nt = jnp.pad(w1t[:3], ((0, 5), (0, 0)))
    w1ct = jnp.pad(w1t[3:], ((0, 5), (0, 0)))
    idx1, u1, v1 = _knn(xt8, x8, w1nt, w1ct, b1.reshape(1, 64))
    x1 = _gmax(u1.reshape(M, 128), idx1.reshape(M * K // 128, 128),
               v1.reshape(M, 64))
    x1 = x1.reshape(B, N, 64)
    x1t = jnp.transpose(x1, (0, 2, 1))

    w2t = jnp.transpose(W2)
    idx2, u2, v2 = _knn(x1, x1t, w2t[:64], w2t[64:], b2.reshape(1, 64))
    x2 = _gmax(u2.reshape(M, 128), idx2.reshape(M * K // 128, 128),
               v2.reshape(M, 64))
    x2t = jnp.transpose(x2.reshape(B, N, 64), (0, 2, 1))

    return _final(x1t, x2t, W5[:, :64], W5[:, 64:], b5.reshape(512, 1))


# speedup vs baseline: 20.8739x; 1.0028x over previous
import functools

import jax
import jax.numpy as jnp
from jax import lax
from jax.experimental import pallas as pl
from jax.experimental.pallas import tpu as pltpu
from jax.experimental.pallas import tpu_sc as plsc

B, N, K = 4, 4096, 16
M = B * N
ROWS = 256
NEG = float("-inf")


def _knn_body(xt_ref, xall_ref, wnt_ref, wct_ref, bias_ref,
              idx_ref, u_ref, v_ref):
    b = pl.program_id(0)
    rows = xt_ref[0]
    alln = xall_ref[0]
    colsq = jnp.sum(alln * alln, axis=1, keepdims=True)
    rows_a = jnp.concatenate([2.0 * rows, jnp.ones_like(rows[:, :1])], axis=1)
    all_a = jnp.concatenate([alln, -colsq], axis=1)
    inner = lax.dot_general(rows_a, all_a, (((1,), (1,)), ((), ())),
                            preferred_element_type=jnp.float32)
    rowsq = jnp.sum(rows * rows, axis=1, keepdims=True)
    d = inner - rowsq

    bi = lax.bitcast_convert_type(d, jnp.int32)
    skey = bi ^ (lax.shift_right_arithmetic(bi, 31) & jnp.int32(0x7FFFFFFF))
    vkey = lax.shift_right_arithmetic(skey, 1) + jnp.int32(0x40000000)
    iota_i = lax.broadcasted_iota(jnp.int32, d.shape, 1)
    fkeys = lax.bitcast_convert_type(
        (vkey & jnp.int32(-4096)) | (jnp.int32(N - 1) - iota_i), jnp.float32)
    cols = []
    for t in range(K):
        m = jnp.max(fkeys, axis=1, keepdims=True)
        mb = lax.bitcast_convert_type(m, jnp.int32)
        cols.append(jnp.int32(N - 1) - (mb & jnp.int32(4095)))
        fkeys = jnp.where(fkeys == m, NEG, fkeys)
    m17 = jnp.max(fkeys, axis=1, keepdims=True)
    mb17 = lax.bitcast_convert_type(m17, jnp.int32)
    col17 = jnp.int32(N - 1) - (mb17 & jnp.int32(4095))
    e16 = jnp.max(jnp.where(iota_i == cols[K - 1], d, NEG), axis=1, keepdims=True)
    e17 = jnp.max(jnp.where(iota_i == col17, d, NEG), axis=1, keepdims=True)
    cols[K - 1] = jnp.where(e17 > e16, col17, cols[K - 1])
    idx_ref[0] = jnp.concatenate(cols, axis=1) + b * N

    u = lax.dot_general(rows, wnt_ref[...], (((1,), (0,)), ((), ())),
                        preferred_element_type=jnp.float32)
    u_ref[0] = jnp.concatenate([u, u], axis=1)
    v_ref[0] = lax.dot_general(rows, wct_ref[...], (((1,), (0,)), ((), ())),
                               preferred_element_type=jnp.float32) + bias_ref[...]


def _knn(xt, wnt, wct, bias):
    c = xt.shape[-1]
    grid = (B, N // ROWS)
    return pl.pallas_call(
        _knn_body,
        grid=grid,
        in_specs=[
            pl.BlockSpec((1, ROWS, c), lambda b, i: (b, i, 0)),
            pl.BlockSpec((1, N, c), lambda b, i: (b, 0, 0)),
            pl.BlockSpec((c, 64), lambda b, i: (0, 0)),
            pl.BlockSpec((c, 64), lambda b, i: (0, 0)),
            pl.BlockSpec((1, 64), lambda b, i: (0, 0)),
        ],
        out_specs=[
            pl.BlockSpec((1, ROWS, K), lambda b, i: (b, i, 0)),
            pl.BlockSpec((1, ROWS, 128), lambda b, i: (b, i, 0)),
            pl.BlockSpec((1, ROWS, 64), lambda b, i: (b, i, 0)),
        ],
        out_shape=[
            jax.ShapeDtypeStruct((B, N, K), jnp.int32),
            jax.ShapeDtypeStruct((B, N, 128), jnp.float32),
            jax.ShapeDtypeStruct((B, N, 64), jnp.float32),
        ],
    )(xt, xt, wnt, wct, bias)


_NC, _NS = 2, 16
_NW = _NC * _NS
_PW = M // _NW
_P = 8
_NG = _PW // _P


def _gmax_body(u_hbm, idx_hbm, v_hbm, o_hbm, idx_v, rows_v, v_v, o_v,
               sg0, sg1, so0, so1):
    wid = lax.axis_index("s") * _NC + lax.axis_index("c")
    base = wid * _PW
    sg = (sg0, sg1)
    so = (so0, so1)

    pltpu.sync_copy(idx_hbm.at[pl.ds(wid * _NG, _NG)], idx_v)

    def fire(g, slot):
        pltpu.async_copy(u_hbm.at[idx_v.at[g]], rows_v.at[slot], sg[slot])
        pltpu.async_copy(v_hbm.at[pl.ds(base + g * _P, _P)], v_v.at[slot],
                         sg[slot])

    def drain(g, slot):
        pltpu.make_async_copy(u_hbm.at[idx_v.at[g]], rows_v.at[slot],
                              sg[slot]).wait()
        pltpu.make_async_copy(v_hbm.at[pl.ds(base + g * _P, _P)],
                              v_v.at[slot], sg[slot]).wait()

    def drain_out(g, slot):
        pltpu.make_async_copy(o_v.at[slot],
                              o_hbm.at[pl.ds(base + g * _P, _P)],
                              so[slot]).wait()

    fire(0, 0)

    def step(g2, carry):
        for slot in (0, 1):
            g = g2 * 2 + slot
            nslot = 1 - slot

            @pl.when(g + 1 < _NG)
            def _():
                fire(g + 1, nslot)

            drain(g, slot)

            @pl.when(g >= 2)
            def _():
                drain_out(g - 2, slot)

            for p in range(_P):
                for cch in range(4):
                    sl = pl.ds(cch * 16, 16)
                    acc = rows_v[slot, p * K, sl]
                    for r in range(1, K):
                        acc = jnp.maximum(acc, rows_v[slot, p * K + r, sl])
                    z = acc + v_v[slot, p, sl]
                    o_v[slot, p, sl] = jnp.maximum(z, 0.2 * z)
            pltpu.async_copy(o_v.at[slot], o_hbm.at[pl.ds(base + g * _P, _P)],
                             so[slot])
        return carry

    lax.fori_loop(0, _NG // 2, step, 0)
    drain_out(_NG - 2, 0)
    drain_out(_NG - 1, 1)


def _gmax(u_flat, idx_rows, v_flat):
    mesh = plsc.VectorSubcoreMesh(core_axis_name="c", subcore_axis_name="s")
    f = functools.partial(
        pl.kernel,
        mesh=mesh,
        out_type=jax.ShapeDtypeStruct((M, 64), jnp.float32),
        scratch_types=[
            pltpu.VMEM((_NG, _P * K), jnp.int32),
            pltpu.VMEM((2, _P * K, 128), jnp.float32),
            pltpu.VMEM((2, _P, 64), jnp.float32),
            pltpu.VMEM((2, _P, 64), jnp.float32),
            pltpu.SemaphoreType.DMA,
            pltpu.SemaphoreType.DMA,
            pltpu.SemaphoreType.DMA,
            pltpu.SemaphoreType.DMA,
        ],
    )(_gmax_body)
    return f(u_flat, idx_rows, v_flat)


_CB = 1024


def _final_body(x1_ref, x2_ref, w5a_ref, w5b_ref, b5_ref, o_ref):
    h = (lax.dot_general(w5a_ref[...], x1_ref[0], (((1,), (1,)), ((), ())),
                         preferred_element_type=jnp.float32)
         + lax.dot_general(w5b_ref[...], x2_ref[0], (((1,), (1,)), ((), ())),
                           preferred_element_type=jnp.float32)
         + b5_ref[...])
    o_ref[0] = jnp.maximum(h, 0.2 * h)


def _final(x1r, x2r, w5a, w5b, b5col):
    grid = (B, N // _CB)
    return pl.pallas_call(
        _final_body,
        grid=grid,
        in_specs=[
            pl.BlockSpec((1, _CB, 64), lambda b, i: (b, i, 0)),
            pl.BlockSpec((1, _CB, 64), lambda b, i: (b, i, 0)),
            pl.BlockSpec((512, 64), lambda b, i: (0, 0)),
            pl.BlockSpec((512, 64), lambda b, i: (0, 0)),
            pl.BlockSpec((512, 1), lambda b, i: (0, 0)),
        ],
        out_specs=pl.BlockSpec((1, 512, _CB), lambda b, i: (b, 0, i)),
        out_shape=jax.ShapeDtypeStruct((B, 512, N), jnp.float32),
    )(x1r, x2r, w5a, w5b, b5col)


def kernel(x, W1, b1, W2, b2, W5, b5):
    xt = jnp.transpose(x, (0, 2, 1))
    xt8 = jnp.pad(xt, ((0, 0), (0, 0), (0, 5)))
    w1t = jnp.transpose(W1)
    w1nt = jnp.pad(w1t[:3], ((0, 5), (0, 0)))
    w1ct = jnp.pad(w1t[3:], ((0, 5), (0, 0)))
    idx1, u1, v1 = _knn(xt8, w1nt, w1ct, b1.reshape(1, 64))
    x1 = _gmax(u1.reshape(M, 128), idx1.reshape(M * K // 128, 128),
               v1.reshape(M, 64))
    x1 = x1.reshape(B, N, 64)

    w2t = jnp.transpose(W2)
    idx2, u2, v2 = _knn(x1, w2t[:64], w2t[64:], b2.reshape(1, 64))
    x2 = _gmax(u2.reshape(M, 128), idx2.reshape(M * K // 128, 128),
               v2.reshape(M, 64))
    x2 = x2.reshape(B, N, 64)

    return _final(x1, x2, W5[:, :64], W5[:, 64:], b5.reshape(512, 1))
